# Initial kernel scaffold; baseline (speedup 1.0000x reference)
import jax
import jax.numpy as jnp
from jax.experimental import pallas as pl


def kernel(x, word_table, pos_table):
    raise NotImplementedError("write your pallas kernel here")


# speedup vs baseline: 1.2333x; 1.2333x over previous
import functools

import jax
import jax.numpy as jnp
from jax import lax
from jax.experimental import pallas as pl
from jax.experimental.pallas import tpu as pltpu
from jax.experimental.pallas import tpu_sc as plsc

_BATCH = 4096
_SEQ = 200
_EMBED = 32
_NW = 32
_ROWS_PER_W = _BATCH // _NW
_CHUNK = 100
_CHUNKS_PER_W = _ROWS_PER_W * _SEQ // _CHUNK
_LOOKUPS_PER_W = _ROWS_PER_W * _SEQ


def _emb_kernel(x_hbm, table_hbm, pos_hbm, out_hbm,
                idx_v, pos_v, rows_v, gsem):
    wid = lax.axis_index("c") * 16 + lax.axis_index("s")
    pltpu.sync_copy(x_hbm.at[wid], idx_v)
    pltpu.sync_copy(pos_hbm, pos_v)
    out_base = wid * _LOOKUPS_PER_W

    def chunk_body(c, _):
        cp1 = pltpu.async_copy(
            table_hbm.at[idx_v.at[2 * c]], rows_v.at[pl.ds(0, _CHUNK)], gsem)
        cp2 = pltpu.async_copy(
            table_hbm.at[idx_v.at[2 * c + 1]], rows_v.at[pl.ds(_CHUNK, _CHUNK)], gsem)
        cp1.wait()
        cp2.wait()

        def add_body(r, _):
            rows_v[r, pl.ds(0, 16)] = rows_v[r, pl.ds(0, 16)] + pos_v[r, pl.ds(0, 16)]
            rows_v[r, pl.ds(16, 16)] = rows_v[r, pl.ds(16, 16)] + pos_v[r, pl.ds(16, 16)]
            return 0

        lax.fori_loop(0, _SEQ, add_body, 0)
        pltpu.sync_copy(rows_v, out_hbm.at[pl.ds(out_base + c * _SEQ, _SEQ)])
        return 0

    lax.fori_loop(0, _ROWS_PER_W, chunk_body, 0)


@jax.jit
def kernel(x, word_table, pos_table):
    B, N = x.shape
    x3 = x.reshape(_NW, _CHUNKS_PER_W, _CHUNK).astype(jnp.int32)
    mesh = plsc.VectorSubcoreMesh(core_axis_name="c", subcore_axis_name="s")
    run = pl.kernel(
        _emb_kernel,
        out_type=jax.ShapeDtypeStruct((B * N, _EMBED), jnp.float32),
        mesh=mesh,
        scratch_types=[
            pltpu.VMEM((_CHUNKS_PER_W, _CHUNK), jnp.int32),
            pltpu.VMEM((_SEQ, _EMBED), jnp.float32),
            pltpu.VMEM((_SEQ, _EMBED), jnp.float32),
            pltpu.SemaphoreType.DMA,
        ],
        compiler_params=pltpu.CompilerParams(use_tc_tiling_on_sc=False),
    )
    out = run(x3, word_table, pos_table)
    return out.reshape(B, N, _EMBED)


# speedup vs baseline: 1.4850x; 1.2041x over previous
import functools

import jax
import jax.numpy as jnp
from jax import lax
from jax.experimental import pallas as pl
from jax.experimental.pallas import tpu as pltpu
from jax.experimental.pallas import tpu_sc as plsc

_BATCH = 4096
_SEQ = 200
_EMBED = 32
_NW = 32
_ROWS_PER_W = _BATCH // _NW
_HALF = _SEQ // 2
_LOOKUPS_PER_W = _ROWS_PER_W * _SEQ
_NBUF = 4


def _emb_kernel(x_hbm, table_hbm, pos_hbm, out_hbm,
                idx_v, pos_v,
                r0, r1, r2, r3,
                g0, g1, g2, g3,
                s0, s1, s2, s3):
    wid = lax.axis_index("c") * 16 + lax.axis_index("s")
    pltpu.sync_copy(x_hbm.at[wid], idx_v)
    pltpu.sync_copy(pos_hbm, pos_v)
    out_base = wid * _LOOKUPS_PER_W
    rows = (r0, r1, r2, r3)
    gsem = (g0, g1, g2, g3)
    ssem = (s0, s1, s2, s3)

    def start_gather(c, rbuf, sem):
        pltpu.async_copy(
            table_hbm.at[idx_v.at[2 * c]], rbuf.at[pl.ds(0, _HALF)], sem)
        pltpu.async_copy(
            table_hbm.at[idx_v.at[2 * c + 1]], rbuf.at[pl.ds(_HALF, _HALF)], sem)

    def wait_gather(rbuf, sem):
        pltpu.make_async_copy(table_hbm.at[pl.ds(0, _SEQ)], rbuf, sem).wait()

    def wait_store(rbuf, sem):
        pltpu.make_async_copy(rbuf, out_hbm.at[pl.ds(out_base, _SEQ)], sem).wait()

    start_gather(0, rows[0], gsem[0])
    start_gather(1, rows[1], gsem[1])

    @pl.loop(0, _ROWS_PER_W // _NBUF)
    def chunk_group(gi):
        for j in range(_NBUF):
            c = _NBUF * gi + j
            nb = (j + 2) % _NBUF
            rbuf = rows[j]

            @pl.when(c + 2 < _ROWS_PER_W)
            def _prefetch():
                @pl.when(c >= 2)
                def _drain():
                    wait_store(rows[nb], ssem[nb])
                start_gather(c + 2, rows[nb], gsem[nb])

            wait_gather(rbuf, gsem[j])

            @plsc.parallel_loop(0, _SEQ, step=1, unroll=8)
            def add_body(r):
                rbuf[r, pl.ds(0, 16)] = rbuf[r, pl.ds(0, 16)] + pos_v[r, pl.ds(0, 16)]
                rbuf[r, pl.ds(16, 16)] = rbuf[r, pl.ds(16, 16)] + pos_v[r, pl.ds(16, 16)]

            pltpu.async_copy(
                rbuf, out_hbm.at[pl.ds(out_base + c * _SEQ, _SEQ)], ssem[j])

    for j in range(_NBUF):
        wait_store(rows[j], ssem[j])


@jax.jit
def kernel(x, word_table, pos_table):
    B, N = x.shape
    x3 = x.reshape(_NW, 2 * _ROWS_PER_W, _HALF).astype(jnp.int32)
    mesh = plsc.VectorSubcoreMesh(core_axis_name="c", subcore_axis_name="s")
    run = pl.kernel(
        _emb_kernel,
        out_type=jax.ShapeDtypeStruct((B * N, _EMBED), jnp.float32),
        mesh=mesh,
        scratch_types=(
            [pltpu.VMEM((2 * _ROWS_PER_W, _HALF), jnp.int32),
             pltpu.VMEM((_SEQ, _EMBED), jnp.float32)]
            + [pltpu.VMEM((_SEQ, _EMBED), jnp.float32) for _ in range(_NBUF)]
            + [pltpu.SemaphoreType.DMA for _ in range(2 * _NBUF)]
        ),
        compiler_params=pltpu.CompilerParams(use_tc_tiling_on_sc=False),
    )
    out = run(x3, word_table, pos_table)
    return out.reshape(B, N, _EMBED)
